# baseline (speedup 1.0000x reference)
import functools

import jax
import jax.numpy as jnp
from jax import lax
from jax.experimental import pallas as pl
from jax.experimental.pallas import tpu as pltpu
from jax.experimental.pallas import tpu_sc as plsc

VOCAB = 1000000
DIM = 64
PDIM = 128
B = 4096
L = 200
N_ROWS = B * L

_info = plsc.get_sparse_core_info()
NC, NS = _info.num_cores, _info.num_subcores
NW = NC * NS
B_PER_W = B // NW

_TBLK = 2048
_TGRID = (VOCAB + _TBLK - 1) // _TBLK


def _transpose_table(table_t):
  def body(in_ref, out_ref):
    out_ref[:, :DIM] = in_ref[...].T

  return pl.pallas_call(
      body,
      grid=(_TGRID,),
      in_specs=[pl.BlockSpec((DIM, _TBLK), lambda i: (0, i))],
      out_specs=pl.BlockSpec((_TBLK, PDIM), lambda i: (i, 0)),
      out_shape=jax.ShapeDtypeStruct((VOCAB, PDIM), jnp.float32),
  )(table_t)


def _transpose_out(p3):
  def body(in_ref, out_ref):
    for q in range(2):
      xt = in_ref[q].T
      out_ref[q] = xt[:DIM]

  return pl.pallas_call(
      body,
      grid=(100, 8),
      in_specs=[pl.BlockSpec((2, 512, PDIM), lambda l2, bb: (l2, bb, 0))],
      out_specs=pl.BlockSpec((2, DIM, 512), lambda l2, bb: (l2, 0, bb)),
      out_shape=jax.ShapeDtypeStruct((L, DIM, B), jnp.float32),
  )(p3)


def _make_kernel():
  mesh = plsc.VectorSubcoreMesh(core_axis_name="c", subcore_axis_name="s")

  @functools.partial(
      pl.kernel,
      mesh=mesh,
      out_type=jax.ShapeDtypeStruct((N_ROWS, PDIM), jnp.float32),
      scratch_types=[
          pltpu.VMEM((B_PER_W, L), jnp.int32),
          pltpu.VMEM((B_PER_W,), jnp.int32),
          pltpu.VMEM((B_PER_W,), jnp.int32),
          pltpu.VMEM((B_PER_W, DIM), jnp.float32),
          pltpu.VMEM((B_PER_W, DIM), jnp.float32),
          pltpu.SemaphoreType.DMA,
          pltpu.SemaphoreType.DMA,
          pltpu.SemaphoreType.DMA,
          pltpu.SemaphoreType.DMA,
      ],
      compiler_params=pltpu.CompilerParams(use_tc_tiling_on_sc=False,
                                           needs_layout_passes=False),
  )
  def k(x_hbm, table_hbm, out_hbm, xb, ich0, ich1, rows0, rows1,
        g0, g1, s0, s1):
    wid = lax.axis_index("s") * NC + lax.axis_index("c")
    b0 = wid * B_PER_W
    pltpu.sync_copy(x_hbm.at[pl.ds(b0, B_PER_W)], xb)

    iota = lax.iota(jnp.int32, 16)

    def assemble(l, ich):
      ls = jnp.full((16,), l, jnp.int32)
      for m in range(8):
        v = plsc.load_gather(xb, [iota + 16 * m, ls])
        ich[pl.ds(16 * m, 16)] = v

    def start_gather(ich, rows, sem):
      pltpu.async_copy(table_hbm.at[ich], rows, sem)

    def wait_gather(ich, rows, sem):
      pltpu.make_async_copy(table_hbm.at[ich], rows, sem).wait()

    def start_store(l, rows, sem):
      pltpu.async_copy(
          rows, out_hbm.at[pl.ds(l * B + b0, B_PER_W), pl.ds(0, DIM)], sem)

    def wait_store(rows, sem):
      pltpu.make_async_copy(
          rows, out_hbm.at[pl.ds(b0, B_PER_W), pl.ds(0, DIM)], sem).wait()

    assemble(0, ich0)
    start_gather(ich0, rows0, g0)
    assemble(1, ich1)
    start_gather(ich1, rows1, g1)

    @pl.loop(0, L // 2)
    def _(j):
      l0 = 2 * j
      wait_gather(ich0, rows0, g0)
      start_store(l0, rows0, s0)

      @pl.when(j < L // 2 - 1)
      def _():
        assemble(l0 + 2, ich0)
        wait_store(rows0, s0)
        start_gather(ich0, rows0, g0)

      wait_gather(ich1, rows1, g1)
      start_store(l0 + 1, rows1, s1)

      @pl.when(j < L // 2 - 1)
      def _():
        assemble(l0 + 3, ich1)
        wait_store(rows1, s1)
        start_gather(ich1, rows1, g1)

    wait_store(rows0, s0)
    wait_store(rows1, s1)

  return k


_gather = _make_kernel()


@jax.jit
def kernel(x, table):
  t128 = _transpose_table(table.T)
  t2 = t128.reshape(2 * VOCAB, DIM)
  p2 = _gather(x.astype(jnp.int32) * 2, t2)
  p3 = p2.reshape(L, B, PDIM)
  out_t = _transpose_out(p3)
  return jnp.transpose(out_t, (2, 0, 1))


# speedup vs baseline: 1.2203x; 1.2203x over previous
import functools

import jax
import jax.numpy as jnp
from jax import lax
from jax.experimental import pallas as pl
from jax.experimental.pallas import tpu as pltpu
from jax.experimental.pallas import tpu_sc as plsc

VOCAB = 1000000
DIM = 64
PDIM = 128
B = 4096
L = 200
N_ROWS = B * L

_info = plsc.get_sparse_core_info()
NC, NS = _info.num_cores, _info.num_subcores
NW = NC * NS
B_PER_W = B // NW

_TBLK = 2048
_TGRID = (VOCAB + _TBLK - 1) // _TBLK


def _transpose_table(table_t):
  def body(in_ref, out_ref):
    out_ref[:, :DIM] = in_ref[...].T

  return pl.pallas_call(
      body,
      grid=(_TGRID,),
      in_specs=[pl.BlockSpec((DIM, _TBLK), lambda i: (0, i))],
      out_specs=pl.BlockSpec((_TBLK, PDIM), lambda i: (i, 0)),
      out_shape=jax.ShapeDtypeStruct((VOCAB, PDIM), jnp.float32),
  )(table_t)


def _transpose_out(p3):
  def body(in_ref, out_ref):
    for q in range(2):
      xt = in_ref[q].T
      out_ref[q] = xt[:DIM]

  return pl.pallas_call(
      body,
      grid=(100, 8),
      in_specs=[pl.BlockSpec((2, 512, PDIM), lambda l2, bb: (l2, bb, 0))],
      out_specs=pl.BlockSpec((2, DIM, 512), lambda l2, bb: (l2, 0, bb)),
      out_shape=jax.ShapeDtypeStruct((L, DIM, B), jnp.float32),
  )(p3)


def _make_kernel():
  mesh = plsc.VectorSubcoreMesh(core_axis_name="c", subcore_axis_name="s")

  @functools.partial(
      pl.kernel,
      mesh=mesh,
      out_type=jax.ShapeDtypeStruct((N_ROWS, PDIM), jnp.float32),
      scratch_types=[
          pltpu.VMEM((B_PER_W, L), jnp.int32),
          pltpu.VMEM((B_PER_W,), jnp.int32),
          pltpu.VMEM((B_PER_W,), jnp.int32),
          pltpu.VMEM((B_PER_W, DIM), jnp.float32),
          pltpu.VMEM((B_PER_W, DIM), jnp.float32),
          pltpu.SemaphoreType.DMA,
          pltpu.SemaphoreType.DMA,
          pltpu.SemaphoreType.DMA,
          pltpu.SemaphoreType.DMA,
      ],
      compiler_params=pltpu.CompilerParams(use_tc_tiling_on_sc=False,
                                           needs_layout_passes=False),
  )
  def k(x_hbm, table_hbm, out_hbm, xb, ich0, ich1, rows0, rows1,
        g0, g1, s0, s1):
    wid = lax.axis_index("s") * NC + lax.axis_index("c")
    b0 = wid * B_PER_W
    pltpu.sync_copy(x_hbm.at[pl.ds(b0, B_PER_W)], xb)

    iota = lax.iota(jnp.int32, 16)

    def assemble(l, ich):
      ls = jnp.full((16,), l, jnp.int32)
      for m in range(8):
        v = plsc.load_gather(xb, [iota + 16 * m, ls])
        ich[pl.ds(16 * m, 16)] = v

    def start_gather(ich, rows, sem):
      pltpu.async_copy(table_hbm.at[ich], rows, sem)

    def wait_gather(ich, rows, sem):
      pltpu.make_async_copy(table_hbm.at[ich], rows, sem).wait()

    def start_store(l, rows, sem):
      pltpu.async_copy(
          rows, out_hbm.at[pl.ds(l * B + b0, B_PER_W), pl.ds(0, DIM)], sem)

    def wait_store(rows, sem):
      pltpu.make_async_copy(
          rows, out_hbm.at[pl.ds(b0, B_PER_W), pl.ds(0, DIM)], sem).wait()

    assemble(0, ich0)
    start_gather(ich0, rows0, g0)
    assemble(1, ich1)
    start_gather(ich1, rows1, g1)

    @pl.loop(0, L // 2)
    def _(j):
      l0 = 2 * j
      wait_gather(ich0, rows0, g0)
      start_store(l0, rows0, s0)

      @pl.when(j < L // 2 - 1)
      def _():
        assemble(l0 + 2, ich0)
        wait_store(rows0, s0)
        start_gather(ich0, rows0, g0)

      wait_gather(ich1, rows1, g1)
      start_store(l0 + 1, rows1, s1)

      @pl.when(j < L // 2 - 1)
      def _():
        assemble(l0 + 3, ich1)
        wait_store(rows1, s1)
        start_gather(ich1, rows1, g1)

    wait_store(rows0, s0)
    wait_store(rows1, s1)

  return k


_gather = _make_kernel()


@jax.jit
def kernel(x, table):
  t128 = jnp.concatenate([table, table], axis=1)
  t2 = t128.reshape(2 * VOCAB, DIM)
  p2 = _gather(x.astype(jnp.int32) * 2, t2)
  p3 = p2.reshape(L, B, PDIM)
  return jnp.transpose(p3[:, :, :DIM], (1, 0, 2))


# speedup vs baseline: 1.5380x; 1.2604x over previous
import functools

import jax
import jax.numpy as jnp
from jax import lax
from jax.experimental import pallas as pl
from jax.experimental.pallas import tpu as pltpu
from jax.experimental.pallas import tpu_sc as plsc

VOCAB = 1000000
DIM = 64
PDIM = 128
B = 4096
L = 200
N_ROWS = B * L

_info = plsc.get_sparse_core_info()
NC, NS = _info.num_cores, _info.num_subcores
NW = NC * NS
B_PER_W = B // NW

_TBLK = 2048
_TGRID = (VOCAB + _TBLK - 1) // _TBLK


def _transpose_table(table_t):
  def body(in_ref, out_ref):
    out_ref[:, :DIM] = in_ref[...].T

  return pl.pallas_call(
      body,
      grid=(_TGRID,),
      in_specs=[pl.BlockSpec((DIM, _TBLK), lambda i: (0, i))],
      out_specs=pl.BlockSpec((_TBLK, PDIM), lambda i: (i, 0)),
      out_shape=jax.ShapeDtypeStruct((VOCAB, PDIM), jnp.float32),
  )(table_t)


def _transpose_out(p3):
  def body(in_ref, out_ref):
    for q in range(2):
      xt = in_ref[q].T
      out_ref[q] = xt[:DIM]

  return pl.pallas_call(
      body,
      grid=(100, 8),
      in_specs=[pl.BlockSpec((2, 512, PDIM), lambda l2, bb: (l2, bb, 0))],
      out_specs=pl.BlockSpec((2, DIM, 512), lambda l2, bb: (l2, 0, bb)),
      out_shape=jax.ShapeDtypeStruct((L, DIM, B), jnp.float32),
  )(p3)


def _make_kernel():
  mesh = plsc.VectorSubcoreMesh(core_axis_name="c", subcore_axis_name="s")

  @functools.partial(
      pl.kernel,
      mesh=mesh,
      out_type=jax.ShapeDtypeStruct((N_ROWS, PDIM), jnp.float32),
      scratch_types=[
          pltpu.VMEM((B_PER_W, L), jnp.int32),
          pltpu.VMEM((B_PER_W,), jnp.int32),
          pltpu.VMEM((B_PER_W,), jnp.int32),
          pltpu.VMEM((B_PER_W, DIM), jnp.float32),
          pltpu.VMEM((B_PER_W, DIM), jnp.float32),
          pltpu.SemaphoreType.DMA,
          pltpu.SemaphoreType.DMA,
          pltpu.SemaphoreType.DMA,
          pltpu.SemaphoreType.DMA,
      ],
      compiler_params=pltpu.CompilerParams(use_tc_tiling_on_sc=False,
                                           needs_layout_passes=False),
  )
  def k(x_hbm, table_hbm, out_hbm, xb, ich0, ich1, rows0, rows1,
        g0, g1, s0, s1):
    wid = lax.axis_index("s") * NC + lax.axis_index("c")
    b0 = wid * B_PER_W
    pltpu.sync_copy(x_hbm.at[pl.ds(b0, B_PER_W)], xb)

    iota = lax.iota(jnp.int32, 16)

    def assemble(l, ich):
      ls = jnp.full((16,), l, jnp.int32)
      for m in range(8):
        v = plsc.load_gather(xb, [iota + 16 * m, ls])
        ich[pl.ds(16 * m, 16)] = v

    def start_gather(ich, rows, sem):
      pltpu.async_copy(table_hbm.at[ich], rows, sem)

    def wait_gather(ich, rows, sem):
      pltpu.make_async_copy(table_hbm.at[ich], rows, sem).wait()

    def start_store(l, rows, sem):
      pltpu.async_copy(
          rows, out_hbm.at[pl.ds(l * B + b0, B_PER_W), pl.ds(0, DIM)], sem)

    def wait_store(rows, sem):
      pltpu.make_async_copy(
          rows, out_hbm.at[pl.ds(b0, B_PER_W), pl.ds(0, DIM)], sem).wait()

    assemble(0, ich0)
    start_gather(ich0, rows0, g0)
    assemble(1, ich1)
    start_gather(ich1, rows1, g1)

    @pl.loop(0, L // 2)
    def _(j):
      l0 = 2 * j
      wait_gather(ich0, rows0, g0)
      start_store(l0, rows0, s0)

      @pl.when(j < L // 2 - 1)
      def _():
        assemble(l0 + 2, ich0)
        wait_store(rows0, s0)
        start_gather(ich0, rows0, g0)

      wait_gather(ich1, rows1, g1)
      start_store(l0 + 1, rows1, s1)

      @pl.when(j < L // 2 - 1)
      def _():
        assemble(l0 + 3, ich1)
        wait_store(rows1, s1)
        start_gather(ich1, rows1, g1)

    wait_store(rows0, s0)
    wait_store(rows1, s1)

  return k


_gather = _make_kernel()


@jax.jit
def kernel(x, table):
  t128 = _transpose_table(table.T)
  t2 = t128.reshape(2 * VOCAB, DIM)
  p2 = _gather(x.astype(jnp.int32) * 2, t2)
  p3 = p2.reshape(L, B, PDIM)
  return jnp.transpose(p3[:, :, :DIM], (1, 0, 2))


# speedup vs baseline: 1.8234x; 1.1856x over previous
import functools

import jax
import jax.numpy as jnp
from jax import lax
from jax.experimental import pallas as pl
from jax.experimental.pallas import tpu as pltpu
from jax.experimental.pallas import tpu_sc as plsc

VOCAB = 1000000
DIM = 64
PDIM = 128
B = 4096
L = 200
N_ROWS = B * L

_info = plsc.get_sparse_core_info()
NC, NS = _info.num_cores, _info.num_subcores
NW = NC * NS
B_PER_W = B // NW

_TBLK = 4096
_TGRID = (VOCAB + _TBLK - 1) // _TBLK


def _transpose_table(table_t):
  def body(in_ref, out_ref):
    out_ref[:, :DIM] = in_ref[...].T

  return pl.pallas_call(
      body,
      grid=(_TGRID,),
      in_specs=[pl.BlockSpec((DIM, _TBLK), lambda i: (0, i))],
      out_specs=pl.BlockSpec((_TBLK, PDIM), lambda i: (i, 0)),
      out_shape=jax.ShapeDtypeStruct((VOCAB, PDIM), jnp.float32),
  )(table_t)


def _transpose_out(p3):
  def body(in_ref, out_ref):
    for q in range(2):
      xt = in_ref[q].T
      out_ref[q] = xt[:DIM]

  return pl.pallas_call(
      body,
      grid=(100, 8),
      in_specs=[pl.BlockSpec((2, 512, PDIM), lambda l2, bb: (l2, bb, 0))],
      out_specs=pl.BlockSpec((2, DIM, 512), lambda l2, bb: (l2, 0, bb)),
      out_shape=jax.ShapeDtypeStruct((L, DIM, B), jnp.float32),
  )(p3)


def _make_kernel():
  mesh = plsc.VectorSubcoreMesh(core_axis_name="c", subcore_axis_name="s")

  @functools.partial(
      pl.kernel,
      mesh=mesh,
      out_type=jax.ShapeDtypeStruct((N_ROWS, PDIM), jnp.float32),
      scratch_types=[
          pltpu.VMEM((B_PER_W, L), jnp.int32),
          pltpu.VMEM((B_PER_W,), jnp.int32),
          pltpu.VMEM((B_PER_W,), jnp.int32),
          pltpu.VMEM((B_PER_W, DIM), jnp.float32),
          pltpu.VMEM((B_PER_W, DIM), jnp.float32),
          pltpu.SemaphoreType.DMA,
          pltpu.SemaphoreType.DMA,
          pltpu.SemaphoreType.DMA,
          pltpu.SemaphoreType.DMA,
      ],
      compiler_params=pltpu.CompilerParams(use_tc_tiling_on_sc=False,
                                           needs_layout_passes=False),
  )
  def k(x_hbm, table_hbm, out_hbm, xb, ich0, ich1, rows0, rows1,
        g0, g1, s0, s1):
    wid = lax.axis_index("s") * NC + lax.axis_index("c")
    b0 = wid * B_PER_W
    pltpu.sync_copy(x_hbm.at[pl.ds(b0, B_PER_W)], xb)

    iota = lax.iota(jnp.int32, 16)

    def assemble(l, ich):
      ls = jnp.full((16,), l, jnp.int32)
      for m in range(8):
        v = plsc.load_gather(xb, [iota + 16 * m, ls])
        ich[pl.ds(16 * m, 16)] = v

    def start_gather(ich, rows, sem):
      pltpu.async_copy(table_hbm.at[ich], rows, sem)

    def wait_gather(ich, rows, sem):
      pltpu.make_async_copy(table_hbm.at[ich], rows, sem).wait()

    def start_store(l, rows, sem):
      pltpu.async_copy(
          rows, out_hbm.at[pl.ds(l * B + b0, B_PER_W), pl.ds(0, DIM)], sem)

    def wait_store(rows, sem):
      pltpu.make_async_copy(
          rows, out_hbm.at[pl.ds(b0, B_PER_W), pl.ds(0, DIM)], sem).wait()

    assemble(0, ich0)
    start_gather(ich0, rows0, g0)
    assemble(1, ich1)
    start_gather(ich1, rows1, g1)

    @pl.loop(0, L // 2)
    def _(j):
      l0 = 2 * j
      wait_gather(ich0, rows0, g0)
      start_store(l0, rows0, s0)

      @pl.when(j < L // 2 - 1)
      def _():
        assemble(l0 + 2, ich0)
        wait_store(rows0, s0)
        start_gather(ich0, rows0, g0)

      wait_gather(ich1, rows1, g1)
      start_store(l0 + 1, rows1, s1)

      @pl.when(j < L // 2 - 1)
      def _():
        assemble(l0 + 3, ich1)
        wait_store(rows1, s1)
        start_gather(ich1, rows1, g1)

    wait_store(rows0, s0)
    wait_store(rows1, s1)

  return k


_gather = _make_kernel()


@jax.jit
def kernel(x, table):
  t128 = _transpose_table(table.T)
  t2 = t128.reshape(2 * VOCAB, DIM)
  p2 = _gather(x.astype(jnp.int32) * 2, t2)
  p3 = p2.reshape(L, B, PDIM)
  return jnp.transpose(p3[:, :, :DIM], (1, 0, 2))


# speedup vs baseline: 2.0398x; 1.1187x over previous
import functools

import jax
import jax.numpy as jnp
from jax import lax
from jax.experimental import pallas as pl
from jax.experimental.pallas import tpu as pltpu
from jax.experimental.pallas import tpu_sc as plsc

VOCAB = 1000000
DIM = 64
PDIM = 128
B = 4096
L = 200
N_ROWS = B * L

_info = plsc.get_sparse_core_info()
NC, NS = _info.num_cores, _info.num_subcores
NW = NC * NS
B_PER_W = B // NW

_TBLK = 8192
_TGRID = (VOCAB + _TBLK - 1) // _TBLK


def _transpose_table(table_t):
  def body(in_ref, out_ref):
    out_ref[:, :DIM] = in_ref[...].T

  return pl.pallas_call(
      body,
      grid=(_TGRID,),
      in_specs=[pl.BlockSpec((DIM, _TBLK), lambda i: (0, i))],
      out_specs=pl.BlockSpec((_TBLK, PDIM), lambda i: (i, 0)),
      out_shape=jax.ShapeDtypeStruct((VOCAB, PDIM), jnp.float32),
  )(table_t)


def _transpose_out(p3):
  def body(in_ref, out_ref):
    for q in range(2):
      xt = in_ref[q].T
      out_ref[q] = xt[:DIM]

  return pl.pallas_call(
      body,
      grid=(100, 8),
      in_specs=[pl.BlockSpec((2, 512, PDIM), lambda l2, bb: (l2, bb, 0))],
      out_specs=pl.BlockSpec((2, DIM, 512), lambda l2, bb: (l2, 0, bb)),
      out_shape=jax.ShapeDtypeStruct((L, DIM, B), jnp.float32),
  )(p3)


def _make_kernel():
  mesh = plsc.VectorSubcoreMesh(core_axis_name="c", subcore_axis_name="s")

  @functools.partial(
      pl.kernel,
      mesh=mesh,
      out_type=jax.ShapeDtypeStruct((N_ROWS, PDIM), jnp.float32),
      scratch_types=[
          pltpu.VMEM((B_PER_W, L), jnp.int32),
          pltpu.VMEM((B_PER_W,), jnp.int32),
          pltpu.VMEM((B_PER_W,), jnp.int32),
          pltpu.VMEM((B_PER_W, DIM), jnp.float32),
          pltpu.VMEM((B_PER_W, DIM), jnp.float32),
          pltpu.SemaphoreType.DMA,
          pltpu.SemaphoreType.DMA,
          pltpu.SemaphoreType.DMA,
          pltpu.SemaphoreType.DMA,
      ],
      compiler_params=pltpu.CompilerParams(use_tc_tiling_on_sc=False,
                                           needs_layout_passes=False),
  )
  def k(x_hbm, table_hbm, out_hbm, xb, ich0, ich1, rows0, rows1,
        g0, g1, s0, s1):
    wid = lax.axis_index("s") * NC + lax.axis_index("c")
    b0 = wid * B_PER_W
    pltpu.sync_copy(x_hbm.at[pl.ds(b0, B_PER_W)], xb)

    iota = lax.iota(jnp.int32, 16)

    def assemble(l, ich):
      ls = jnp.full((16,), l, jnp.int32)
      for m in range(8):
        v = plsc.load_gather(xb, [iota + 16 * m, ls])
        ich[pl.ds(16 * m, 16)] = v

    def start_gather(ich, rows, sem):
      pltpu.async_copy(table_hbm.at[ich], rows, sem)

    def wait_gather(ich, rows, sem):
      pltpu.make_async_copy(table_hbm.at[ich], rows, sem).wait()

    def start_store(l, rows, sem):
      pltpu.async_copy(
          rows, out_hbm.at[pl.ds(l * B + b0, B_PER_W), pl.ds(0, DIM)], sem)

    def wait_store(rows, sem):
      pltpu.make_async_copy(
          rows, out_hbm.at[pl.ds(b0, B_PER_W), pl.ds(0, DIM)], sem).wait()

    assemble(0, ich0)
    start_gather(ich0, rows0, g0)
    assemble(1, ich1)
    start_gather(ich1, rows1, g1)

    @pl.loop(0, L // 2)
    def _(j):
      l0 = 2 * j
      wait_gather(ich0, rows0, g0)
      start_store(l0, rows0, s0)

      @pl.when(j < L // 2 - 1)
      def _():
        assemble(l0 + 2, ich0)
        wait_store(rows0, s0)
        start_gather(ich0, rows0, g0)

      wait_gather(ich1, rows1, g1)
      start_store(l0 + 1, rows1, s1)

      @pl.when(j < L // 2 - 1)
      def _():
        assemble(l0 + 3, ich1)
        wait_store(rows1, s1)
        start_gather(ich1, rows1, g1)

    wait_store(rows0, s0)
    wait_store(rows1, s1)

  return k


_gather = _make_kernel()


@jax.jit
def kernel(x, table):
  t128 = _transpose_table(table.T)
  t2 = t128.reshape(2 * VOCAB, DIM)
  p2 = _gather(x.astype(jnp.int32) * 2, t2)
  p3 = p2.reshape(L, B, PDIM)
  return jnp.transpose(p3[:, :, :DIM], (1, 0, 2))


# speedup vs baseline: 2.1037x; 1.0313x over previous
import functools

import jax
import jax.numpy as jnp
from jax import lax
from jax.experimental import pallas as pl
from jax.experimental.pallas import tpu as pltpu
from jax.experimental.pallas import tpu_sc as plsc

VOCAB = 1000000
DIM = 64
PDIM = 128
B = 4096
L = 200
N_ROWS = B * L

_info = plsc.get_sparse_core_info()
NC, NS = _info.num_cores, _info.num_subcores
NW = NC * NS
B_PER_W = B // NW

_TBLK = 16384
_TGRID = (VOCAB + _TBLK - 1) // _TBLK


def _transpose_table(table_t):
  def body(in_ref, out_ref):
    out_ref[:, :DIM] = in_ref[...].T

  return pl.pallas_call(
      body,
      grid=(_TGRID,),
      in_specs=[pl.BlockSpec((DIM, _TBLK), lambda i: (0, i))],
      out_specs=pl.BlockSpec((_TBLK, PDIM), lambda i: (i, 0)),
      out_shape=jax.ShapeDtypeStruct((VOCAB, PDIM), jnp.float32),
  )(table_t)


def _transpose_out(p3):
  def body(in_ref, out_ref):
    for q in range(2):
      xt = in_ref[q].T
      out_ref[q] = xt[:DIM]

  return pl.pallas_call(
      body,
      grid=(100, 8),
      in_specs=[pl.BlockSpec((2, 512, PDIM), lambda l2, bb: (l2, bb, 0))],
      out_specs=pl.BlockSpec((2, DIM, 512), lambda l2, bb: (l2, 0, bb)),
      out_shape=jax.ShapeDtypeStruct((L, DIM, B), jnp.float32),
  )(p3)


def _make_kernel():
  mesh = plsc.VectorSubcoreMesh(core_axis_name="c", subcore_axis_name="s")

  @functools.partial(
      pl.kernel,
      mesh=mesh,
      out_type=jax.ShapeDtypeStruct((N_ROWS, PDIM), jnp.float32),
      scratch_types=[
          pltpu.VMEM((B_PER_W, L), jnp.int32),
          pltpu.VMEM((B_PER_W,), jnp.int32),
          pltpu.VMEM((B_PER_W,), jnp.int32),
          pltpu.VMEM((B_PER_W, DIM), jnp.float32),
          pltpu.VMEM((B_PER_W, DIM), jnp.float32),
          pltpu.SemaphoreType.DMA,
          pltpu.SemaphoreType.DMA,
          pltpu.SemaphoreType.DMA,
          pltpu.SemaphoreType.DMA,
      ],
      compiler_params=pltpu.CompilerParams(use_tc_tiling_on_sc=False,
                                           needs_layout_passes=False),
  )
  def k(x_hbm, table_hbm, out_hbm, xb, ich0, ich1, rows0, rows1,
        g0, g1, s0, s1):
    wid = lax.axis_index("s") * NC + lax.axis_index("c")
    b0 = wid * B_PER_W
    pltpu.sync_copy(x_hbm.at[pl.ds(b0, B_PER_W)], xb)

    iota = lax.iota(jnp.int32, 16)

    def assemble(l, ich):
      ls = jnp.full((16,), l, jnp.int32)
      for m in range(8):
        v = plsc.load_gather(xb, [iota + 16 * m, ls])
        ich[pl.ds(16 * m, 16)] = v

    def start_gather(ich, rows, sem):
      pltpu.async_copy(table_hbm.at[ich], rows, sem)

    def wait_gather(ich, rows, sem):
      pltpu.make_async_copy(table_hbm.at[ich], rows, sem).wait()

    def start_store(l, rows, sem):
      pltpu.async_copy(
          rows, out_hbm.at[pl.ds(l * B + b0, B_PER_W), pl.ds(0, DIM)], sem)

    def wait_store(rows, sem):
      pltpu.make_async_copy(
          rows, out_hbm.at[pl.ds(b0, B_PER_W), pl.ds(0, DIM)], sem).wait()

    assemble(0, ich0)
    start_gather(ich0, rows0, g0)
    assemble(1, ich1)
    start_gather(ich1, rows1, g1)

    @pl.loop(0, L // 2)
    def _(j):
      l0 = 2 * j
      wait_gather(ich0, rows0, g0)
      start_store(l0, rows0, s0)

      @pl.when(j < L // 2 - 1)
      def _():
        assemble(l0 + 2, ich0)
        wait_store(rows0, s0)
        start_gather(ich0, rows0, g0)

      wait_gather(ich1, rows1, g1)
      start_store(l0 + 1, rows1, s1)

      @pl.when(j < L // 2 - 1)
      def _():
        assemble(l0 + 3, ich1)
        wait_store(rows1, s1)
        start_gather(ich1, rows1, g1)

    wait_store(rows0, s0)
    wait_store(rows1, s1)

  return k


_gather = _make_kernel()


@jax.jit
def kernel(x, table):
  t128 = _transpose_table(table.T)
  t2 = t128.reshape(2 * VOCAB, DIM)
  p2 = _gather(x.astype(jnp.int32) * 2, t2)
  p3 = p2.reshape(L, B, PDIM)
  return jnp.transpose(p3[:, :, :DIM], (1, 0, 2))


# speedup vs baseline: 2.1324x; 1.0136x over previous
import functools

import jax
import jax.numpy as jnp
from jax import lax
from jax.experimental import pallas as pl
from jax.experimental.pallas import tpu as pltpu
from jax.experimental.pallas import tpu_sc as plsc

VOCAB = 1000000
DIM = 64
PDIM = 128
B = 4096
L = 200
N_ROWS = B * L

_info = plsc.get_sparse_core_info()
NC, NS = _info.num_cores, _info.num_subcores
NW = NC * NS
B_PER_W = B // NW

_TBLK = 32768
_TGRID = (VOCAB + _TBLK - 1) // _TBLK


def _transpose_table(table_t):
  def body(in_ref, out_ref):
    out_ref[:, :DIM] = in_ref[...].T

  return pl.pallas_call(
      body,
      grid=(_TGRID,),
      in_specs=[pl.BlockSpec((DIM, _TBLK), lambda i: (0, i))],
      out_specs=pl.BlockSpec((_TBLK, PDIM), lambda i: (i, 0)),
      out_shape=jax.ShapeDtypeStruct((VOCAB, PDIM), jnp.float32),
  )(table_t)


def _transpose_out(p3):
  def body(in_ref, out_ref):
    for q in range(2):
      xt = in_ref[q].T
      out_ref[q] = xt[:DIM]

  return pl.pallas_call(
      body,
      grid=(100, 8),
      in_specs=[pl.BlockSpec((2, 512, PDIM), lambda l2, bb: (l2, bb, 0))],
      out_specs=pl.BlockSpec((2, DIM, 512), lambda l2, bb: (l2, 0, bb)),
      out_shape=jax.ShapeDtypeStruct((L, DIM, B), jnp.float32),
  )(p3)


def _make_kernel():
  mesh = plsc.VectorSubcoreMesh(core_axis_name="c", subcore_axis_name="s")

  @functools.partial(
      pl.kernel,
      mesh=mesh,
      out_type=jax.ShapeDtypeStruct((N_ROWS, PDIM), jnp.float32),
      scratch_types=[
          pltpu.VMEM((B_PER_W, L), jnp.int32),
          pltpu.VMEM((B_PER_W,), jnp.int32),
          pltpu.VMEM((B_PER_W,), jnp.int32),
          pltpu.VMEM((B_PER_W, DIM), jnp.float32),
          pltpu.VMEM((B_PER_W, DIM), jnp.float32),
          pltpu.SemaphoreType.DMA,
          pltpu.SemaphoreType.DMA,
          pltpu.SemaphoreType.DMA,
          pltpu.SemaphoreType.DMA,
      ],
      compiler_params=pltpu.CompilerParams(use_tc_tiling_on_sc=False,
                                           needs_layout_passes=False),
  )
  def k(x_hbm, table_hbm, out_hbm, xb, ich0, ich1, rows0, rows1,
        g0, g1, s0, s1):
    wid = lax.axis_index("s") * NC + lax.axis_index("c")
    b0 = wid * B_PER_W
    pltpu.sync_copy(x_hbm.at[pl.ds(b0, B_PER_W)], xb)

    iota = lax.iota(jnp.int32, 16)

    def assemble(l, ich):
      ls = jnp.full((16,), l, jnp.int32)
      for m in range(8):
        v = plsc.load_gather(xb, [iota + 16 * m, ls])
        ich[pl.ds(16 * m, 16)] = v

    def start_gather(ich, rows, sem):
      pltpu.async_copy(table_hbm.at[ich], rows, sem)

    def wait_gather(ich, rows, sem):
      pltpu.make_async_copy(table_hbm.at[ich], rows, sem).wait()

    def start_store(l, rows, sem):
      pltpu.async_copy(
          rows, out_hbm.at[pl.ds(l * B + b0, B_PER_W), pl.ds(0, DIM)], sem)

    def wait_store(rows, sem):
      pltpu.make_async_copy(
          rows, out_hbm.at[pl.ds(b0, B_PER_W), pl.ds(0, DIM)], sem).wait()

    assemble(0, ich0)
    start_gather(ich0, rows0, g0)
    assemble(1, ich1)
    start_gather(ich1, rows1, g1)

    @pl.loop(0, L // 2)
    def _(j):
      l0 = 2 * j
      wait_gather(ich0, rows0, g0)
      start_store(l0, rows0, s0)

      @pl.when(j < L // 2 - 1)
      def _():
        assemble(l0 + 2, ich0)
        wait_store(rows0, s0)
        start_gather(ich0, rows0, g0)

      wait_gather(ich1, rows1, g1)
      start_store(l0 + 1, rows1, s1)

      @pl.when(j < L // 2 - 1)
      def _():
        assemble(l0 + 3, ich1)
        wait_store(rows1, s1)
        start_gather(ich1, rows1, g1)

    wait_store(rows0, s0)
    wait_store(rows1, s1)

  return k


_gather = _make_kernel()


@jax.jit
def kernel(x, table):
  t128 = _transpose_table(table.T)
  t2 = t128.reshape(2 * VOCAB, DIM)
  p2 = _gather(x.astype(jnp.int32) * 2, t2)
  p3 = p2.reshape(L, B, PDIM)
  return jnp.transpose(p3[:, :, :DIM], (1, 0, 2))
